# baseline (speedup 1.0000x reference)
import functools

import jax
import jax.numpy as jnp
from jax import lax
from jax.experimental import pallas as pl
from jax.experimental.pallas import tpu as pltpu
from jax.experimental.pallas import tpu_sc as plsc

VOCAB_SIZES = (6, 7, 12, 7, 96)
EMB_DIMS = (3, 4, 6, 4, 50)
N_ROWS = 16384
X_COLS = 64
OUT_COLS = X_COLS + sum(EMB_DIMS)

NUM_CORES = 2
NUM_SUBCORES = 16
NUM_WORKERS = NUM_CORES * NUM_SUBCORES
ROWS_PER_W = N_ROWS // NUM_WORKERS
LANES = 16
CHUNKS = ROWS_PER_W // LANES

_COL_OFF = []
_acc = X_COLS
for _d in EMB_DIMS:
    _COL_OFF.append(_acc)
    _acc += _d


def _body(x_hbm, y_hbm, e0, e1, e2, e3, e4, out_hbm,
          x_v, y_v, t0, t1, t2, t3, t4, out_v, sem):
    tabs = (t0, t1, t2, t3, t4)
    wid = lax.axis_index("s") * NUM_CORES + lax.axis_index("c")
    base = wid * ROWS_PER_W

    cps = [
        pltpu.make_async_copy(
            x_hbm.at[pl.ds(base * X_COLS, ROWS_PER_W * X_COLS)], x_v, sem),
        pltpu.make_async_copy(
            y_hbm.at[pl.ds(base * 5, ROWS_PER_W * 5)], y_v, sem),
    ] + [
        pltpu.make_async_copy(e, t, sem)
        for e, t in zip((e0, e1, e2, e3, e4), tabs)
    ]
    for cp in cps:
        cp.start()
    for cp in cps:
        cp.wait()

    iota = lax.broadcasted_iota(jnp.int32, (LANES,), 0)

    def chunk(c, carry):
        rows = c * LANES + iota
        rx = rows * X_COLS
        ro = rows * OUT_COLS
        for j in range(X_COLS):
            val = plsc.load_gather(x_v, [rx + j])
            plsc.store_scatter(out_v, [ro + j], val)
        ry = rows * 5
        for i in range(5):
            yi = plsc.load_gather(y_v, [ry + i])
            addr = yi * EMB_DIMS[i]
            for cc in range(EMB_DIMS[i]):
                val = plsc.load_gather(tabs[i], [addr + cc])
                plsc.store_scatter(out_v, [ro + (_COL_OFF[i] + cc)], val)
        return carry

    lax.fori_loop(0, CHUNKS, chunk, 0)
    pltpu.sync_copy(out_v, out_hbm.at[pl.ds(base * OUT_COLS,
                                            ROWS_PER_W * OUT_COLS)])


_feature_call = functools.partial(
    pl.kernel,
    out_type=jax.ShapeDtypeStruct((N_ROWS * OUT_COLS,), jnp.float32),
    mesh=plsc.VectorSubcoreMesh(core_axis_name="c", subcore_axis_name="s"),
    compiler_params=pltpu.CompilerParams(needs_layout_passes=False),
    scratch_types=[
        pltpu.VMEM((ROWS_PER_W * X_COLS,), jnp.float32),
        pltpu.VMEM((ROWS_PER_W * 5,), jnp.int32),
        pltpu.VMEM((VOCAB_SIZES[0] * EMB_DIMS[0],), jnp.float32),
        pltpu.VMEM((VOCAB_SIZES[1] * EMB_DIMS[1],), jnp.float32),
        pltpu.VMEM((VOCAB_SIZES[2] * EMB_DIMS[2],), jnp.float32),
        pltpu.VMEM((VOCAB_SIZES[3] * EMB_DIMS[3],), jnp.float32),
        pltpu.VMEM((VOCAB_SIZES[4] * EMB_DIMS[4],), jnp.float32),
        pltpu.VMEM((ROWS_PER_W * OUT_COLS,), jnp.float32),
        pltpu.SemaphoreType.DMA,
    ],
)(_body)


def kernel(x, y, E0, E1, E2, E3, E4):
    out_flat = _feature_call(
        jnp.reshape(x, (-1,)), jnp.reshape(y, (-1,)),
        jnp.reshape(E0, (-1,)), jnp.reshape(E1, (-1,)),
        jnp.reshape(E2, (-1,)), jnp.reshape(E3, (-1,)),
        jnp.reshape(E4, (-1,)))
    return jnp.reshape(out_flat, (N_ROWS, OUT_COLS))


# speedup vs baseline: 1.0912x; 1.0912x over previous
import functools

import jax
import jax.numpy as jnp
from jax import lax
from jax.experimental import pallas as pl
from jax.experimental.pallas import tpu as pltpu
from jax.experimental.pallas import tpu_sc as plsc

VOCAB_SIZES = (6, 7, 12, 7, 96)
EMB_DIMS = (3, 4, 6, 4, 50)
N_ROWS = 16384
X_COLS = 64
OUT_COLS = X_COLS + sum(EMB_DIMS)

NUM_CORES = 2
NUM_SUBCORES = 16
NUM_WORKERS = NUM_CORES * NUM_SUBCORES
ROWS_PER_W = N_ROWS // NUM_WORKERS
LANES = 16
CHUNKS = ROWS_PER_W // LANES

_COL_OFF = []
_acc = X_COLS
for _d in EMB_DIMS:
    _COL_OFF.append(_acc)
    _acc += _d


def _body(x_hbm, y_hbm, e0, e1, e2, e3, e4, out_hbm,
          y_v, t0, t1, t2, t3, t4, out_v, sem):
    tabs = (t0, t1, t2, t3, t4)
    wid = lax.axis_index("s") * NUM_CORES + lax.axis_index("c")
    base = wid * ROWS_PER_W

    cps = [
        pltpu.make_async_copy(
            x_hbm.at[pl.ds(base, ROWS_PER_W)], out_v.at[:, pl.ds(0, X_COLS)],
            sem),
        pltpu.make_async_copy(y_hbm.at[pl.ds(base, ROWS_PER_W)], y_v, sem),
    ] + [
        pltpu.make_async_copy(e, t, sem)
        for e, t in zip((e0, e1, e2, e3, e4), tabs)
    ]
    for cp in cps:
        cp.start()
    for cp in cps:
        cp.wait()

    iota = lax.broadcasted_iota(jnp.int32, (LANES,), 0)

    def chunk(c, carry):
        rows = c * LANES + iota
        for i in range(5):
            yi = plsc.load_gather(y_v, [rows, jnp.full((LANES,), i, jnp.int32)])
            for cc in range(EMB_DIMS[i]):
                val = plsc.load_gather(
                    tabs[i], [yi, jnp.full((LANES,), cc, jnp.int32)])
                plsc.store_scatter(
                    out_v,
                    [rows, jnp.full((LANES,), _COL_OFF[i] + cc, jnp.int32)],
                    val)
        return carry

    lax.fori_loop(0, CHUNKS, chunk, 0)
    pltpu.sync_copy(out_v, out_hbm.at[pl.ds(base, ROWS_PER_W)])


_feature_call = functools.partial(
    pl.kernel,
    out_type=jax.ShapeDtypeStruct((N_ROWS, OUT_COLS), jnp.float32),
    mesh=plsc.VectorSubcoreMesh(core_axis_name="c", subcore_axis_name="s"),
    compiler_params=pltpu.CompilerParams(
        needs_layout_passes=False, use_tc_tiling_on_sc=False),
    scratch_types=[
        pltpu.VMEM((ROWS_PER_W, 5), jnp.int32),
        pltpu.VMEM((VOCAB_SIZES[0], EMB_DIMS[0]), jnp.float32),
        pltpu.VMEM((VOCAB_SIZES[1], EMB_DIMS[1]), jnp.float32),
        pltpu.VMEM((VOCAB_SIZES[2], EMB_DIMS[2]), jnp.float32),
        pltpu.VMEM((VOCAB_SIZES[3], EMB_DIMS[3]), jnp.float32),
        pltpu.VMEM((VOCAB_SIZES[4], EMB_DIMS[4]), jnp.float32),
        pltpu.VMEM((ROWS_PER_W, OUT_COLS), jnp.float32),
        pltpu.SemaphoreType.DMA,
    ],
)(_body)


def kernel(x, y, E0, E1, E2, E3, E4):
    return _feature_call(x, y, E0, E1, E2, E3, E4)


# speedup vs baseline: 3.5292x; 3.2342x over previous
import functools

import jax
import jax.numpy as jnp
from jax import lax
from jax.experimental import pallas as pl
from jax.experimental.pallas import tpu as pltpu
from jax.experimental.pallas import tpu_sc as plsc

VOCAB_SIZES = (6, 7, 12, 7, 96)
EMB_DIMS = (3, 4, 6, 4, 50)
N_ROWS = 16384
X_COLS = 64
OUT_COLS = X_COLS + sum(EMB_DIMS)

NUM_CORES = 2
NUM_SUBCORES = 16
NUM_WORKERS = NUM_CORES * NUM_SUBCORES
COLS_PER_W = N_ROWS // NUM_WORKERS
LANES = 16
CHUNKS = COLS_PER_W // LANES

_COL_OFF = []
_acc = X_COLS
for _d in EMB_DIMS:
    _COL_OFF.append(_acc)
    _acc += _d


def _body(xt_hbm, yt_hbm, e0, e1, e2, e3, e4, out_hbm, y_v, t0, t1, t2, t3, t4,
          o_v, sem):
    tabs = (t0, t1, t2, t3, t4)
    wid = lax.axis_index("s") * NUM_CORES + lax.axis_index("c")
    base = wid * COLS_PER_W

    cps = [
        pltpu.make_async_copy(
            xt_hbm.at[:, pl.ds(base, COLS_PER_W)],
            o_v.at[pl.ds(0, X_COLS), :], sem),
        pltpu.make_async_copy(yt_hbm.at[:, pl.ds(base, COLS_PER_W)], y_v, sem),
    ] + [
        pltpu.make_async_copy(e, t, sem)
        for e, t in zip((e0, e1, e2, e3, e4), tabs)
    ]
    for cp in cps:
        cp.start()
    for cp in cps:
        cp.wait()

    def chunk(k, carry):
        s = k * LANES
        for i in range(5):
            yi = y_v[i, pl.ds(s, LANES)]
            for cc in range(EMB_DIMS[i]):
                val = plsc.load_gather(
                    tabs[i], [jnp.full((LANES,), cc, jnp.int32), yi])
                o_v[_COL_OFF[i] + cc, pl.ds(s, LANES)] = val
        return carry

    lax.fori_loop(0, CHUNKS, chunk, 0)
    pltpu.sync_copy(o_v, out_hbm.at[:, pl.ds(base, COLS_PER_W)])


_feature_call = functools.partial(
    pl.kernel,
    out_type=jax.ShapeDtypeStruct((OUT_COLS, N_ROWS), jnp.float32),
    mesh=plsc.VectorSubcoreMesh(core_axis_name="c", subcore_axis_name="s"),
    compiler_params=pltpu.CompilerParams(
        needs_layout_passes=False, use_tc_tiling_on_sc=True),
    scratch_types=[
        pltpu.VMEM((5, COLS_PER_W), jnp.int32),
        pltpu.VMEM((EMB_DIMS[0], VOCAB_SIZES[0]), jnp.float32),
        pltpu.VMEM((EMB_DIMS[1], VOCAB_SIZES[1]), jnp.float32),
        pltpu.VMEM((EMB_DIMS[2], VOCAB_SIZES[2]), jnp.float32),
        pltpu.VMEM((EMB_DIMS[3], VOCAB_SIZES[3]), jnp.float32),
        pltpu.VMEM((EMB_DIMS[4], VOCAB_SIZES[4]), jnp.float32),
        pltpu.VMEM((OUT_COLS, COLS_PER_W), jnp.float32),
        pltpu.SemaphoreType.DMA,
    ],
)(_body)


def kernel(x, y, E0, E1, E2, E3, E4):
    out_t = _feature_call(x.T, y.T, E0.T, E1.T, E2.T, E3.T, E4.T)
    return out_t.T
